# Initial kernel scaffold; baseline (speedup 1.0000x reference)
import functools

import jax
import jax.numpy as jnp
from jax import lax
from jax.experimental import pallas as pl
from jax.experimental.pallas import tpu as pltpu
from jax.experimental.pallas import tpu_sc as plsc

NC, NS = 2, 16
NW = NC * NS
L = 16
R = 16
B_PAD = 30720
ROWS_PW = B_PAD // NW
NCHUNK = ROWS_PW // R


def _sc_gather_sum(idx_prep, memory_table, K, D):
    mesh = plsc.VectorSubcoreMesh(core_axis_name="c", subcore_axis_name="s")

    @functools.partial(
        pl.kernel,
        mesh=mesh,
        out_type=jax.ShapeDtypeStruct((B_PAD, D), jnp.float32),
        scratch_types=[
            pltpu.VMEM((NCHUNK, K, R), jnp.int32),
            pltpu.VMEM((2, K * R, D), jnp.float32),
            pltpu.VMEM((2, R, D), jnp.float32),
            pltpu.SemaphoreType.DMA,
            pltpu.SemaphoreType.DMA,
            pltpu.SemaphoreType.DMA,
            pltpu.SemaphoreType.DMA,
        ],
    )
    def k(idx_hbm, table_hbm, out_hbm, idx_v, gbuf, ostg, gsem0, gsem1,
          osem0, osem1):
        wid = lax.axis_index("s") * NC + lax.axis_index("c")
        cbase = wid * NCHUNK
        gsems = (gsem0, gsem1)
        osems = (osem0, osem1)

        pltpu.sync_copy(idx_hbm.at[pl.ds(cbase, NCHUNK)], idx_v)

        def fire(slot, cc):
            for kk in range(K):
                pltpu.async_copy(
                    table_hbm.at[idx_v.at[cc, kk]],
                    gbuf.at[slot].at[pl.ds(kk * R, R)],
                    gsems[slot],
                )

        def drain_gathers(slot):
            pltpu.make_async_copy(
                table_hbm.at[pl.ds(0, K * R)], gbuf.at[slot], gsems[slot]
            ).wait()

        def wait_ostg(slot):
            pltpu.make_async_copy(
                table_hbm.at[pl.ds(0, R)], ostg.at[slot], osems[slot]
            ).wait()

        def compute(slot):
            @pl.loop(0, R)
            def _(i):
                for j in range(D // L):
                    v = gbuf[slot, i, pl.ds(L * j, L)]
                    for kk in range(1, K):
                        v = v + gbuf[slot, kk * R + i, pl.ds(L * j, L)]
                    ostg[slot, i, pl.ds(L * j, L)] = v

        def put(slot, cc):
            pltpu.async_copy(
                ostg.at[slot],
                out_hbm.at[pl.ds((cbase + cc) * R, R)],
                osems[slot],
            )

        for s in range(2):
            pltpu.async_copy(table_hbm.at[pl.ds(0, R)], ostg.at[s], osems[s])
        fire(0, 0)

        @pl.loop(0, (NCHUNK - 2) // 2)
        def _(p):
            for s in range(2):
                cc = 2 * p + s
                fire(1 - s, cc + 1)
                drain_gathers(s)
                wait_ostg(s)
                compute(s)
                put(s, cc)

        fire(1, NCHUNK - 1)
        drain_gathers(0)
        wait_ostg(0)
        compute(0)
        put(0, NCHUNK - 2)
        drain_gathers(1)
        wait_ostg(1)
        compute(1)
        put(1, NCHUNK - 1)
        wait_ostg(0)
        wait_ostg(1)

    return k(idx_prep, memory_table)


def _tc_dense(raw, agg, W_msg, W_nbr, bias, B, D_RAW, D_NBR, D_MSG):
    BM = 512

    def body(raw_ref, agg_ref, wm_ref, wn_ref, b_ref, o_ref):
        acc = jnp.dot(raw_ref[...], wm_ref[...],
                      preferred_element_type=jnp.float32)
        acc = acc + jnp.dot(agg_ref[...], wn_ref[...],
                            preferred_element_type=jnp.float32)
        o_ref[...] = jnp.maximum(acc + b_ref[...], 0.0)

    return pl.pallas_call(
        body,
        grid=(pl.cdiv(B, BM),),
        in_specs=[
            pl.BlockSpec((BM, D_RAW), lambda i: (i, 0)),
            pl.BlockSpec((BM, D_NBR), lambda i: (i, 0)),
            pl.BlockSpec((D_RAW, D_MSG), lambda i: (0, 0)),
            pl.BlockSpec((D_NBR, D_MSG), lambda i: (0, 0)),
            pl.BlockSpec((1, D_MSG), lambda i: (0, 0)),
        ],
        out_specs=pl.BlockSpec((BM, D_MSG), lambda i: (i, 0)),
        out_shape=jax.ShapeDtypeStruct((B, D_MSG), jnp.float32),
    )(raw, agg, W_msg, W_nbr, bias)


def kernel(raw_messages, neighbors, memory_table, W_msg, b_msg, W_nbr, b_nbr):
    b, K = neighbors.shape
    D_NBR = memory_table.shape[1]
    D_RAW = raw_messages.shape[1]
    D_MSG = W_msg.shape[1]

    nbr_pad = jnp.pad(neighbors, ((0, B_PAD - b), (0, 0)))
    idx_prep = nbr_pad.reshape(B_PAD // R, R, K).transpose(0, 2, 1)
    agg = _sc_gather_sum(idx_prep, memory_table, K, D_NBR)
    bias = (b_msg + b_nbr).reshape(1, D_MSG)
    return _tc_dense(raw_messages, agg, W_msg, W_nbr, bias,
                     b, D_RAW, D_NBR, D_MSG)


# speedup vs baseline: 2.5469x; 2.5469x over previous
import functools

import jax
import jax.numpy as jnp
from jax import lax
from jax.experimental import pallas as pl
from jax.experimental.pallas import tpu as pltpu
from jax.experimental.pallas import tpu_sc as plsc

NC, NS = 2, 16
NW = NC * NS
L = 16
R = 16
B_PAD = 30720
ROWS_PW = B_PAD // NW
NCHUNK = ROWS_PW // R


def _sc_gather_sum(idx_prep, memory_table, K, D):
    mesh = plsc.VectorSubcoreMesh(core_axis_name="c", subcore_axis_name="s")

    @functools.partial(
        pl.kernel,
        mesh=mesh,
        compiler_params=pltpu.CompilerParams(use_tc_tiling_on_sc=False),
        out_type=jax.ShapeDtypeStruct((B_PAD, D), jnp.float32),
        scratch_types=[
            pltpu.VMEM((NCHUNK, K, R), jnp.int32),
            pltpu.VMEM((2, K * R, D), jnp.float32),
            pltpu.VMEM((2, R, D), jnp.float32),
            pltpu.SemaphoreType.DMA,
            pltpu.SemaphoreType.DMA,
            pltpu.SemaphoreType.DMA,
            pltpu.SemaphoreType.DMA,
        ],
    )
    def k(idx_hbm, table_hbm, out_hbm, idx_v, gbuf, ostg, gsem0, gsem1,
          osem0, osem1):
        wid = lax.axis_index("s") * NC + lax.axis_index("c")
        cbase = wid * NCHUNK
        gsems = (gsem0, gsem1)
        osems = (osem0, osem1)

        pltpu.sync_copy(idx_hbm.at[pl.ds(cbase, NCHUNK)], idx_v)

        def fire(slot, cc):
            for kk in range(K):
                pltpu.async_copy(
                    table_hbm.at[idx_v.at[cc, kk]],
                    gbuf.at[slot].at[pl.ds(kk * R, R)],
                    gsems[slot],
                )

        def drain_gathers(slot):
            pltpu.make_async_copy(
                table_hbm.at[pl.ds(0, K * R)], gbuf.at[slot], gsems[slot]
            ).wait()

        def wait_ostg(slot):
            pltpu.make_async_copy(
                table_hbm.at[pl.ds(0, R)], ostg.at[slot], osems[slot]
            ).wait()

        def compute(slot):
            @pl.loop(0, R)
            def _(i):
                for j in range(D // L):
                    v = gbuf[slot, i, pl.ds(L * j, L)]
                    for kk in range(1, K):
                        v = v + gbuf[slot, kk * R + i, pl.ds(L * j, L)]
                    ostg[slot, i, pl.ds(L * j, L)] = v

        def put(slot, cc):
            pltpu.async_copy(
                ostg.at[slot],
                out_hbm.at[pl.ds((cbase + cc) * R, R)],
                osems[slot],
            )

        for s in range(2):
            pltpu.async_copy(table_hbm.at[pl.ds(0, R)], ostg.at[s], osems[s])
        fire(0, 0)

        @pl.loop(0, (NCHUNK - 2) // 2)
        def _(p):
            for s in range(2):
                cc = 2 * p + s
                fire(1 - s, cc + 1)
                drain_gathers(s)
                wait_ostg(s)
                compute(s)
                put(s, cc)

        fire(1, NCHUNK - 1)
        drain_gathers(0)
        wait_ostg(0)
        compute(0)
        put(0, NCHUNK - 2)
        drain_gathers(1)
        wait_ostg(1)
        compute(1)
        put(1, NCHUNK - 1)
        wait_ostg(0)
        wait_ostg(1)

    return k(idx_prep, memory_table)


def _tc_dense(raw, agg, W_msg, W_nbr, bias, B, D_RAW, D_NBR, D_MSG):
    BM = 512

    def body(raw_ref, agg_ref, wm_ref, wn_ref, b_ref, o_ref):
        acc = jnp.dot(raw_ref[...], wm_ref[...],
                      preferred_element_type=jnp.float32)
        acc = acc + jnp.dot(agg_ref[...], wn_ref[...],
                            preferred_element_type=jnp.float32)
        o_ref[...] = jnp.maximum(acc + b_ref[...], 0.0)

    return pl.pallas_call(
        body,
        grid=(pl.cdiv(B, BM),),
        in_specs=[
            pl.BlockSpec((BM, D_RAW), lambda i: (i, 0)),
            pl.BlockSpec((BM, D_NBR), lambda i: (i, 0)),
            pl.BlockSpec((D_RAW, D_MSG), lambda i: (0, 0)),
            pl.BlockSpec((D_NBR, D_MSG), lambda i: (0, 0)),
            pl.BlockSpec((1, D_MSG), lambda i: (0, 0)),
        ],
        out_specs=pl.BlockSpec((BM, D_MSG), lambda i: (i, 0)),
        out_shape=jax.ShapeDtypeStruct((B, D_MSG), jnp.float32),
    )(raw, agg, W_msg, W_nbr, bias)


def kernel(raw_messages, neighbors, memory_table, W_msg, b_msg, W_nbr, b_nbr):
    b, K = neighbors.shape
    D_NBR = memory_table.shape[1]
    D_RAW = raw_messages.shape[1]
    D_MSG = W_msg.shape[1]

    nbr_pad = jnp.pad(neighbors, ((0, B_PAD - b), (0, 0)))
    idx_prep = nbr_pad.reshape(B_PAD // R, R, K).transpose(0, 2, 1)
    agg = _sc_gather_sum(idx_prep, memory_table, K, D_NBR)
    bias = (b_msg + b_nbr).reshape(1, D_MSG)
    return _tc_dense(raw_messages, agg, W_msg, W_nbr, bias,
                     b, D_RAW, D_NBR, D_MSG)
